# baseline (speedup 1.0000x reference)
import jax
import jax.numpy as jnp
from jax.experimental import pallas as pl
from jax.experimental.pallas import tpu as pltpu

_BLOCK_ROWS = 512


def _add_pe_kernel(x_ref, pe_ref, o_ref):
    o_ref[...] = x_ref[...] + pe_ref[...][None, :, :]


def kernel(x, pe_weight):
    batch, seq_len, embed_dim = x.shape
    pe = pe_weight[:seq_len]
    grid = (seq_len // _BLOCK_ROWS,)
    return pl.pallas_call(
        _add_pe_kernel,
        grid=grid,
        in_specs=[
            pl.BlockSpec((batch, _BLOCK_ROWS, embed_dim), lambda i: (0, i, 0)),
            pl.BlockSpec((_BLOCK_ROWS, embed_dim), lambda i: (i, 0)),
        ],
        out_specs=pl.BlockSpec((batch, _BLOCK_ROWS, embed_dim), lambda i: (0, i, 0)),
        out_shape=jax.ShapeDtypeStruct(x.shape, x.dtype),
        compiler_params=pltpu.CompilerParams(
            dimension_semantics=("arbitrary",),
        ),
    )(x, pe)


# speedup vs baseline: 1.0094x; 1.0094x over previous
import jax
import jax.numpy as jnp
from jax.experimental import pallas as pl
from jax.experimental.pallas import tpu as pltpu

_BLOCK_ROWS = 1024


def _add_pe_kernel(x_ref, pe_ref, o_ref):
    o_ref[...] = x_ref[...] + pe_ref[...][None, :, :]


def kernel(x, pe_weight):
    batch, seq_len, embed_dim = x.shape
    pe = pe_weight[:seq_len]
    block_batch = 2
    grid = (seq_len // _BLOCK_ROWS, batch // block_batch)
    return pl.pallas_call(
        _add_pe_kernel,
        grid=grid,
        in_specs=[
            pl.BlockSpec((block_batch, _BLOCK_ROWS, embed_dim), lambda i, b: (b, i, 0)),
            pl.BlockSpec((_BLOCK_ROWS, embed_dim), lambda i, b: (i, 0)),
        ],
        out_specs=pl.BlockSpec((block_batch, _BLOCK_ROWS, embed_dim), lambda i, b: (b, i, 0)),
        out_shape=jax.ShapeDtypeStruct(x.shape, x.dtype),
        compiler_params=pltpu.CompilerParams(
            dimension_semantics=("arbitrary", "arbitrary"),
        ),
    )(x, pe)


# speedup vs baseline: 1.0107x; 1.0013x over previous
import jax
import jax.numpy as jnp
from jax.experimental import pallas as pl
from jax.experimental.pallas import tpu as pltpu

_BLOCK_ROWS = 2048


def _add_pe_kernel(x_ref, pe_ref, o_ref):
    o_ref[...] = x_ref[...] + pe_ref[...][None, :, :]


def kernel(x, pe_weight):
    batch, seq_len, embed_dim = x.shape
    pe = pe_weight[:seq_len]
    block_batch = 1
    grid = (seq_len // _BLOCK_ROWS, batch // block_batch)
    return pl.pallas_call(
        _add_pe_kernel,
        grid=grid,
        in_specs=[
            pl.BlockSpec((block_batch, _BLOCK_ROWS, embed_dim), lambda i, b: (b, i, 0)),
            pl.BlockSpec((_BLOCK_ROWS, embed_dim), lambda i, b: (i, 0)),
        ],
        out_specs=pl.BlockSpec((block_batch, _BLOCK_ROWS, embed_dim), lambda i, b: (b, i, 0)),
        out_shape=jax.ShapeDtypeStruct(x.shape, x.dtype),
        compiler_params=pltpu.CompilerParams(
            dimension_semantics=("arbitrary", "arbitrary"),
        ),
    )(x, pe)
